# Initial kernel scaffold; baseline (speedup 1.0000x reference)
import jax
import jax.numpy as jnp
from jax.experimental import pallas as pl


def kernel(q, k, v, W_proj, b_proj):
    raise NotImplementedError("write your pallas kernel here")


# speedup vs baseline: 1.3022x; 1.3022x over previous
import functools
import math

import jax
import jax.numpy as jnp
from jax.experimental import pallas as pl
from jax.experimental.pallas import tpu as pltpu

BLKQ, BLKK = 64, 32
TOPK_FRAC = 0.1


def _stats_kernel(q_ref, k_ref, v_ref, km_ref, kvsum_ref, ksum_ref, lut_ref,
                  *, mb, nb, topk):
    q = q_ref[0]
    k = k_ref[0]
    v = v_ref[0]
    l, d = q.shape
    km = jnp.mean(k, axis=0, keepdims=True)
    km_ref[0] = km
    pq = jnp.mean(q.reshape(mb, BLKQ, d), axis=1)
    pk = jnp.mean(k.reshape(nb, BLKK, d), axis=1) - km
    ps = jax.lax.dot_general(pq, pk, (((1,), (1,)), ((), ())),
                             precision=jax.lax.Precision.HIGHEST,
                             preferred_element_type=jnp.float32)
    col = jax.lax.broadcasted_iota(jnp.int32, ps.shape, 1)
    for t in range(topk):
        mx = jnp.max(ps, axis=1, keepdims=True)
        idx = jnp.min(jnp.where(ps >= mx, col, nb), axis=1, keepdims=True)
        lut_ref[0, :, t:t + 1] = idx
        ps = jnp.where(col == idx, -jnp.inf, ps)
    kf = jax.nn.softmax(k, axis=-1)
    kvsum_ref[0] = jax.lax.dot_general(kf, v, (((0,), (0,)), ((), ())),
                                       preferred_element_type=jnp.float32)
    ksum_ref[0] = jnp.sum(kf, axis=0, keepdims=True)


def _attn_kernel(lut_ref, q_ref, k_ref, v_ref, km_ref, kvsum_ref, ksum_ref,
                 w_ref, b_ref, o_ref, kc_scr, vc_scr, *, topk, scale):
    h = pl.program_id(0)
    m = pl.program_id(1)
    km = km_ref[0]
    for t in range(topk):
        idx = lut_ref[h, m, t]
        off = idx * BLKK
        kc_scr[t * BLKK:(t + 1) * BLKK, :] = k_ref[0, pl.ds(off, BLKK), :] - km
        vc_scr[t * BLKK:(t + 1) * BLKK, :] = v_ref[0, pl.ds(off, BLKK), :]
    qb = q_ref[0]
    s = jax.lax.dot_general(qb, kc_scr[...], (((1,), (1,)), ((), ())),
                            preferred_element_type=jnp.float32) * scale
    s = s - jnp.max(s, axis=1, keepdims=True)
    p = jnp.exp(s)
    p = p / jnp.sum(p, axis=1, keepdims=True)
    o_s = jax.lax.dot_general(p, vc_scr[...], (((1,), (0,)), ((), ())),
                              preferred_element_type=jnp.float32)
    qf = jax.nn.softmax(qb, axis=-1)
    denom = jnp.sum(qf * ksum_ref[0], axis=1, keepdims=True) + 1e-6
    num = jax.lax.dot_general(qf, kvsum_ref[0], (((1,), (0,)), ((), ())),
                              preferred_element_type=jnp.float32)
    o_l = num / denom
    o_l = jax.lax.dot_general(o_l, w_ref[...], (((1,), (1,)), ((), ())),
                              preferred_element_type=jnp.float32) + b_ref[0]
    o_ref[0] = o_l + o_s


def kernel(q, k, v, W_proj, b_proj):
    b, l, h, d = q.shape
    bh = b * h
    mb = l // BLKQ
    nb = l // BLKK
    topk = min(nb, int(TOPK_FRAC * nb))
    scale = 1.0 / math.sqrt(d)

    qt = jnp.transpose(q, (0, 2, 1, 3)).reshape(bh, l, d)
    kt = jnp.transpose(k, (0, 2, 1, 3)).reshape(bh, l, d)
    vt = jnp.transpose(v, (0, 2, 1, 3)).reshape(bh, l, d)

    km, kvsum, ksum, lut = pl.pallas_call(
        functools.partial(_stats_kernel, mb=mb, nb=nb, topk=topk),
        grid=(bh,),
        in_specs=[
            pl.BlockSpec((1, l, d), lambda i: (i, 0, 0)),
            pl.BlockSpec((1, l, d), lambda i: (i, 0, 0)),
            pl.BlockSpec((1, l, d), lambda i: (i, 0, 0)),
        ],
        out_specs=[
            pl.BlockSpec((1, 1, d), lambda i: (i, 0, 0)),
            pl.BlockSpec((1, d, d), lambda i: (i, 0, 0)),
            pl.BlockSpec((1, 1, d), lambda i: (i, 0, 0)),
            pl.BlockSpec((1, mb, topk), lambda i: (i, 0, 0)),
        ],
        out_shape=[
            jax.ShapeDtypeStruct((bh, 1, d), jnp.float32),
            jax.ShapeDtypeStruct((bh, d, d), jnp.float32),
            jax.ShapeDtypeStruct((bh, 1, d), jnp.float32),
            jax.ShapeDtypeStruct((bh, mb, topk), jnp.int32),
        ],
        compiler_params=pltpu.CompilerParams(
            dimension_semantics=("arbitrary",)),
    )(qt, kt, vt)

    out = pl.pallas_call(
        functools.partial(_attn_kernel, topk=topk, scale=scale),
        grid=(bh, mb),
        in_specs=[
            pl.BlockSpec(memory_space=pltpu.SMEM),
            pl.BlockSpec((1, BLKQ, d), lambda i, j: (i, j, 0)),
            pl.BlockSpec((1, l, d), lambda i, j: (i, 0, 0)),
            pl.BlockSpec((1, l, d), lambda i, j: (i, 0, 0)),
            pl.BlockSpec((1, 1, d), lambda i, j: (i, 0, 0)),
            pl.BlockSpec((1, d, d), lambda i, j: (i, 0, 0)),
            pl.BlockSpec((1, 1, d), lambda i, j: (i, 0, 0)),
            pl.BlockSpec((d, d), lambda i, j: (0, 0)),
            pl.BlockSpec((1, d), lambda i, j: (0, 0)),
        ],
        out_specs=pl.BlockSpec((1, BLKQ, d), lambda i, j: (i, j, 0)),
        out_shape=jax.ShapeDtypeStruct((bh, l, d), jnp.float32),
        scratch_shapes=[
            pltpu.VMEM((topk * BLKK, d), jnp.float32),
            pltpu.VMEM((topk * BLKK, d), jnp.float32),
        ],
        compiler_params=pltpu.CompilerParams(
            dimension_semantics=("arbitrary", "arbitrary")),
    )(lut, qt, kt, vt, km, kvsum, ksum, W_proj, b_proj.reshape(1, d))

    return jnp.transpose(out.reshape(b, h, l, d), (0, 2, 1, 3))


# speedup vs baseline: 1.9697x; 1.5126x over previous
import functools
import math

import jax
import jax.numpy as jnp
from jax.experimental import pallas as pl
from jax.experimental.pallas import tpu as pltpu

BLKQ, BLKK = 64, 32
TOPK_FRAC = 0.1


def _stats_kernel(q_ref, k_ref, v_ref, km_ref, kvsum_ref, ksum_ref, lut_ref,
                  *, mb, nb, topk):
    q = q_ref[0]
    k = k_ref[0]
    v = v_ref[0]
    l, d = q.shape
    km = jnp.mean(k, axis=0, keepdims=True)
    km_ref[0] = km
    pq = jnp.mean(q.reshape(mb, BLKQ, d), axis=1)
    pk = jnp.mean(k.reshape(nb, BLKK, d), axis=1) - km
    ps = jax.lax.dot_general(pq, pk, (((1,), (1,)), ((), ())),
                             precision=jax.lax.Precision.HIGHEST,
                             preferred_element_type=jnp.float32)
    col = jax.lax.broadcasted_iota(jnp.int32, ps.shape, 1)
    for t in range(topk):
        mx = jnp.max(ps, axis=1, keepdims=True)
        idx = jnp.min(jnp.where(ps >= mx, col, nb), axis=1, keepdims=True)
        lut_ref[0, :, t:t + 1] = idx
        ps = jnp.where(col == idx, -jnp.inf, ps)
    kf = jax.nn.softmax(k, axis=-1)
    kvsum_ref[0] = jax.lax.dot_general(kf, v, (((0,), (0,)), ((), ())),
                                       preferred_element_type=jnp.float32)
    ksum_ref[0] = jnp.sum(kf, axis=0, keepdims=True)


def _attn_kernel(lut_ref, q_ref, k_ref, v_ref, kvsum_ref, ksum_ref,
                 w_ref, b_ref, o_ref, kc_scr, vc_scr, *, topk, scale, mg):
    h = pl.program_id(0)
    jg = pl.program_id(1)
    ks = ksum_ref[0]
    kv = kvsum_ref[0]
    w = w_ref[...]
    bb = b_ref[0]
    for g in range(mg):
        m = jg * mg + g
        for t in range(topk):
            idx = lut_ref[h, m, t]
            off = idx * BLKK
            kc_scr[g, t * BLKK:(t + 1) * BLKK, :] = k_ref[0, pl.ds(off, BLKK), :]
            vc_scr[g, t * BLKK:(t + 1) * BLKK, :] = v_ref[0, pl.ds(off, BLKK), :]
    for g in range(mg):
        qb = q_ref[0, g * BLKQ:(g + 1) * BLKQ, :]
        qbs = qb * scale
        s = jax.lax.dot_general(qbs, kc_scr[g], (((1,), (1,)), ((), ())),
                                preferred_element_type=jnp.float32)
        s = s - jnp.max(s, axis=1, keepdims=True)
        p = jnp.exp(s)
        rs = jnp.sum(p, axis=1, keepdims=True)
        o_s = jax.lax.dot_general(p, vc_scr[g], (((1,), (0,)), ((), ())),
                                  preferred_element_type=jnp.float32) / rs
        qf = jax.nn.softmax(qb, axis=-1)
        denom = jnp.sum(qf * ks, axis=1, keepdims=True) + 1e-6
        num = jax.lax.dot_general(qf, kv, (((1,), (0,)), ((), ())),
                                  preferred_element_type=jnp.float32)
        o_l = num / denom
        o_l = jax.lax.dot_general(o_l, w, (((1,), (1,)), ((), ())),
                                  preferred_element_type=jnp.float32) + bb
        o_ref[0, g * BLKQ:(g + 1) * BLKQ, :] = o_l + o_s


def kernel(q, k, v, W_proj, b_proj):
    b, l, h, d = q.shape
    bh = b * h
    mb = l // BLKQ
    nb = l // BLKK
    topk = min(nb, int(TOPK_FRAC * nb))
    scale = 1.0 / math.sqrt(d)

    qt = jnp.transpose(q, (0, 2, 1, 3)).reshape(bh, l, d)
    kt = jnp.transpose(k, (0, 2, 1, 3)).reshape(bh, l, d)
    vt = jnp.transpose(v, (0, 2, 1, 3)).reshape(bh, l, d)

    km, kvsum, ksum, lut = pl.pallas_call(
        functools.partial(_stats_kernel, mb=mb, nb=nb, topk=topk),
        grid=(bh,),
        in_specs=[
            pl.BlockSpec((1, l, d), lambda i: (i, 0, 0)),
            pl.BlockSpec((1, l, d), lambda i: (i, 0, 0)),
            pl.BlockSpec((1, l, d), lambda i: (i, 0, 0)),
        ],
        out_specs=[
            pl.BlockSpec((1, 1, d), lambda i: (i, 0, 0)),
            pl.BlockSpec((1, d, d), lambda i: (i, 0, 0)),
            pl.BlockSpec((1, 1, d), lambda i: (i, 0, 0)),
            pl.BlockSpec((1, mb, topk), lambda i: (i, 0, 0)),
        ],
        out_shape=[
            jax.ShapeDtypeStruct((bh, 1, d), jnp.float32),
            jax.ShapeDtypeStruct((bh, d, d), jnp.float32),
            jax.ShapeDtypeStruct((bh, 1, d), jnp.float32),
            jax.ShapeDtypeStruct((bh, mb, topk), jnp.int32),
        ],
        compiler_params=pltpu.CompilerParams(
            dimension_semantics=("arbitrary",)),
    )(qt, kt, vt)

    mg = 8
    out = pl.pallas_call(
        functools.partial(_attn_kernel, topk=topk, scale=scale, mg=mg),
        grid=(bh, mb // mg),
        in_specs=[
            pl.BlockSpec(memory_space=pltpu.SMEM),
            pl.BlockSpec((1, mg * BLKQ, d), lambda i, j: (i, j, 0)),
            pl.BlockSpec((1, l, d), lambda i, j: (i, 0, 0)),
            pl.BlockSpec((1, l, d), lambda i, j: (i, 0, 0)),
            pl.BlockSpec((1, d, d), lambda i, j: (i, 0, 0)),
            pl.BlockSpec((1, 1, d), lambda i, j: (i, 0, 0)),
            pl.BlockSpec((d, d), lambda i, j: (0, 0)),
            pl.BlockSpec((1, d), lambda i, j: (0, 0)),
        ],
        out_specs=pl.BlockSpec((1, mg * BLKQ, d), lambda i, j: (i, j, 0)),
        out_shape=jax.ShapeDtypeStruct((bh, l, d), jnp.float32),
        scratch_shapes=[
            pltpu.VMEM((mg, topk * BLKK, d), jnp.float32),
            pltpu.VMEM((mg, topk * BLKK, d), jnp.float32),
        ],
        compiler_params=pltpu.CompilerParams(
            dimension_semantics=("arbitrary", "arbitrary")),
    )(lut, qt, kt, vt, kvsum, ksum, W_proj, b_proj.reshape(1, d))

    return jnp.transpose(out.reshape(b, h, l, d), (0, 2, 1, 3))


# speedup vs baseline: 2.0120x; 1.0214x over previous
import functools
import math

import jax
import jax.numpy as jnp
from jax.experimental import pallas as pl
from jax.experimental.pallas import tpu as pltpu

BLKQ, BLKK = 64, 32
TOPK_FRAC = 0.1


def _stats_kernel(q_ref, k_ref, v_ref, km_ref, kvsum_ref, ksum_ref, lut_ref,
                  *, mb, nb, topk):
    q = q_ref[0]
    k = k_ref[0]
    v = v_ref[0]
    l, d = q.shape
    km = jnp.mean(k, axis=0, keepdims=True)
    km_ref[0] = km
    pq = jnp.mean(q.reshape(mb, BLKQ, d), axis=1)
    pk = jnp.mean(k.reshape(nb, BLKK, d), axis=1) - km
    ps = jax.lax.dot_general(pq, pk, (((1,), (1,)), ((), ())),
                             precision=jax.lax.Precision.HIGHEST,
                             preferred_element_type=jnp.float32)
    col = jax.lax.broadcasted_iota(jnp.int32, ps.shape, 1)
    for t in range(topk):
        mx = jnp.max(ps, axis=1, keepdims=True)
        idx = jnp.min(jnp.where(ps >= mx, col, nb), axis=1, keepdims=True)
        lut_ref[0, :, t:t + 1] = idx
        ps = jnp.where(col == idx, -jnp.inf, ps)
    kf = jax.nn.softmax(k, axis=-1)
    kvsum_ref[0] = jax.lax.dot_general(kf, v, (((0,), (0,)), ((), ())),
                                       preferred_element_type=jnp.float32)
    ksum_ref[0] = jnp.sum(kf, axis=0, keepdims=True)


def _attn_kernel(lut_ref, q_ref, k_ref, v_ref, kvsum_ref, ksum_ref,
                 w_ref, b_ref, o_ref, kc_scr, vc_scr, *, topk, scale, mg):
    h = pl.program_id(0)
    jg = pl.program_id(1)
    ks = ksum_ref[0]
    kv = kvsum_ref[0]
    w = w_ref[...]
    bb = b_ref[0]
    for g in range(mg):
        m = jg * mg + g
        for t in range(topk):
            idx = lut_ref[h, m, t]
            off = idx * BLKK
            kc_scr[g, t * BLKK:(t + 1) * BLKK, :] = k_ref[0, pl.ds(off, BLKK), :]
            vc_scr[g, t * BLKK:(t + 1) * BLKK, :] = v_ref[0, pl.ds(off, BLKK), :]
    for g in range(mg):
        qb = q_ref[0, g * BLKQ:(g + 1) * BLKQ, :]
        qbs = qb * scale
        s = jax.lax.dot_general(qbs, kc_scr[g], (((1,), (1,)), ((), ())),
                                preferred_element_type=jnp.float32)
        s = s - jnp.max(s, axis=1, keepdims=True)
        p = jnp.exp(s)
        rs = jnp.sum(p, axis=1, keepdims=True)
        o_s = jax.lax.dot_general(p, vc_scr[g], (((1,), (0,)), ((), ())),
                                  preferred_element_type=jnp.float32) / rs
        qf = jax.nn.softmax(qb, axis=-1)
        denom = jnp.sum(qf * ks, axis=1, keepdims=True) + 1e-6
        num = jax.lax.dot_general(qf, kv, (((1,), (0,)), ((), ())),
                                  preferred_element_type=jnp.float32)
        o_l = num / denom
        o_l = jax.lax.dot_general(o_l, w, (((1,), (1,)), ((), ())),
                                  preferred_element_type=jnp.float32) + bb
        o_ref[0, g * BLKQ:(g + 1) * BLKQ, :] = o_l + o_s


def kernel(q, k, v, W_proj, b_proj):
    b, l, h, d = q.shape
    bh = b * h
    mb = l // BLKQ
    nb = l // BLKK
    topk = min(nb, int(TOPK_FRAC * nb))
    scale = 1.0 / math.sqrt(d)

    qt = jnp.transpose(q, (0, 2, 1, 3)).reshape(bh, l, d)
    kt = jnp.transpose(k, (0, 2, 1, 3)).reshape(bh, l, d)
    vt = jnp.transpose(v, (0, 2, 1, 3)).reshape(bh, l, d)

    km, kvsum, ksum, lut = pl.pallas_call(
        functools.partial(_stats_kernel, mb=mb, nb=nb, topk=topk),
        grid=(bh,),
        in_specs=[
            pl.BlockSpec((1, l, d), lambda i: (i, 0, 0)),
            pl.BlockSpec((1, l, d), lambda i: (i, 0, 0)),
            pl.BlockSpec((1, l, d), lambda i: (i, 0, 0)),
        ],
        out_specs=[
            pl.BlockSpec((1, 1, d), lambda i: (i, 0, 0)),
            pl.BlockSpec((1, d, d), lambda i: (i, 0, 0)),
            pl.BlockSpec((1, 1, d), lambda i: (i, 0, 0)),
            pl.BlockSpec((1, mb, topk), lambda i: (i, 0, 0)),
        ],
        out_shape=[
            jax.ShapeDtypeStruct((bh, 1, d), jnp.float32),
            jax.ShapeDtypeStruct((bh, d, d), jnp.float32),
            jax.ShapeDtypeStruct((bh, 1, d), jnp.float32),
            jax.ShapeDtypeStruct((bh, mb, topk), jnp.int32),
        ],
        compiler_params=pltpu.CompilerParams(
            dimension_semantics=("arbitrary",)),
    )(qt, kt, vt)

    mg = 16
    out = pl.pallas_call(
        functools.partial(_attn_kernel, topk=topk, scale=scale, mg=mg),
        grid=(bh, mb // mg),
        in_specs=[
            pl.BlockSpec(memory_space=pltpu.SMEM),
            pl.BlockSpec((1, mg * BLKQ, d), lambda i, j: (i, j, 0)),
            pl.BlockSpec((1, l, d), lambda i, j: (i, 0, 0)),
            pl.BlockSpec((1, l, d), lambda i, j: (i, 0, 0)),
            pl.BlockSpec((1, d, d), lambda i, j: (i, 0, 0)),
            pl.BlockSpec((1, 1, d), lambda i, j: (i, 0, 0)),
            pl.BlockSpec((d, d), lambda i, j: (0, 0)),
            pl.BlockSpec((1, d), lambda i, j: (0, 0)),
        ],
        out_specs=pl.BlockSpec((1, mg * BLKQ, d), lambda i, j: (i, j, 0)),
        out_shape=jax.ShapeDtypeStruct((bh, l, d), jnp.float32),
        scratch_shapes=[
            pltpu.VMEM((mg, topk * BLKK, d), jnp.float32),
            pltpu.VMEM((mg, topk * BLKK, d), jnp.float32),
        ],
        compiler_params=pltpu.CompilerParams(
            dimension_semantics=("arbitrary", "arbitrary")),
    )(lut, qt, kt, vt, kvsum, ksum, W_proj, b_proj.reshape(1, d))

    return jnp.transpose(out.reshape(b, h, l, d), (0, 2, 1, 3))
